# baseline (speedup 1.0000x reference)
import functools

import jax
import jax.numpy as jnp
from jax import lax
from jax.experimental import pallas as pl
from jax.experimental.pallas import tpu as pltpu
from jax.experimental.pallas import tpu_sc as plsc

N_USERS = 1000000
N_ITEMS = 1000000
EMBED_DIM = 32
BATCH = 16384

NUM_CORES = 2
NUM_SUBCORES = 16
NUM_WORKERS = NUM_CORES * NUM_SUBCORES
B_PER_W = BATCH // NUM_WORKERS
CHUNK = 128
NCHUNK = B_PER_W // CHUNK
LANES = 16
GROUPS = B_PER_W // LANES
SROW = LANES + 1


def _body(user_hbm, item_hbm, u_hbm, i_hbm, out_hbm,
          idx_u, idx_i, rows_u, rows_i, psum, out_v, sem):
  wid = lax.axis_index("s") * NUM_CORES + lax.axis_index("c")
  base = wid * B_PER_W

  for j in range(NCHUNK):
    pltpu.sync_copy(u_hbm.at[pl.ds(base + j * CHUNK, CHUNK)], idx_u.at[j])
    pltpu.sync_copy(i_hbm.at[pl.ds(base + j * CHUNK, CHUNK)], idx_i.at[j])

  copies = []
  for j in range(NCHUNK):
    copies.append(pltpu.async_copy(
        user_hbm.at[idx_u.at[j]], rows_u.at[pl.ds(j * CHUNK, CHUNK)], sem))
    copies.append(pltpu.async_copy(
        item_hbm.at[idx_i.at[j]], rows_i.at[pl.ds(j * CHUNK, CHUNK)], sem))
  for c in copies:
    c.wait()

  row_iota = lax.iota(jnp.int32, LANES)

  @pl.loop(0, GROUPS)
  def _group(g):
    row0 = g * LANES
    for r in range(LANES):
      u0 = rows_u[row0 + r, pl.ds(0, LANES)]
      u1 = rows_u[row0 + r, pl.ds(LANES, LANES)]
      v0 = rows_i[row0 + r, pl.ds(0, LANES)]
      v1 = rows_i[row0 + r, pl.ds(LANES, LANES)]
      psum[r, pl.ds(0, LANES)] = u0 * v0 + u1 * v1
    acc = plsc.load_gather(psum, [row_iota, jnp.zeros((LANES,), jnp.int32)])
    for col in range(1, LANES):
      acc = acc + plsc.load_gather(
          psum, [row_iota, jnp.full((LANES,), col, jnp.int32)])
    out_v[pl.ds(row0, LANES)] = acc

  pltpu.sync_copy(out_v, out_hbm.at[pl.ds(base, B_PER_W)])


@functools.partial(
    pl.kernel,
    out_type=jax.ShapeDtypeStruct((BATCH,), jnp.float32),
    mesh=plsc.VectorSubcoreMesh(
        core_axis_name="c", subcore_axis_name="s",
        num_cores=NUM_CORES, num_subcores=NUM_SUBCORES),
    compiler_params=pltpu.CompilerParams(
        needs_layout_passes=False, use_tc_tiling_on_sc=False),
    scratch_types=[
        pltpu.VMEM((NCHUNK, CHUNK), jnp.int32),
        pltpu.VMEM((NCHUNK, CHUNK), jnp.int32),
        pltpu.VMEM((B_PER_W, EMBED_DIM), jnp.float32),
        pltpu.VMEM((B_PER_W, EMBED_DIM), jnp.float32),
        pltpu.VMEM((LANES, SROW), jnp.float32),
        pltpu.VMEM((B_PER_W,), jnp.float32),
        pltpu.SemaphoreType.DMA,
    ],
)
def _sc_dot(user_hbm, item_hbm, u_hbm, i_hbm, out_hbm,
            idx_u, idx_i, rows_u, rows_i, psum, out_v, sem):
  _body(user_hbm, item_hbm, u_hbm, i_hbm, out_hbm,
        idx_u, idx_i, rows_u, rows_i, psum, out_v, sem)


def kernel(u, i, user_table, item_table):
  return _sc_dot(user_table, item_table,
                 u.astype(jnp.int32), i.astype(jnp.int32))


# speedup vs baseline: 2.7676x; 2.7676x over previous
import functools

import jax
import jax.numpy as jnp
from jax import lax
from jax.experimental import pallas as pl
from jax.experimental.pallas import tpu as pltpu
from jax.experimental.pallas import tpu_sc as plsc

EMBED_DIM = 32
BATCH = 16384
TILE = 128

NUM_CORES = 2
NUM_SUBCORES = 16
NUM_WORKERS = NUM_CORES * NUM_SUBCORES
B_PER_W = BATCH // NUM_WORKERS
LANES = 16
HALF = LANES
GROUPS = B_PER_W // LANES
SROW = LANES + 1


def _body(user_hbm, item_hbm, u_hbm, i_hbm, out_hbm,
          u_vm, i_vm, ub0, ub1, ib0, ib1, psum, out_v,
          su0, su1, si0, si1):
  wid = lax.axis_index("s") * NUM_CORES + lax.axis_index("c")
  base = wid * B_PER_W

  pltpu.sync_copy(u_hbm.at[pl.ds(base, B_PER_W)], u_vm)
  pltpu.sync_copy(i_hbm.at[pl.ds(base, B_PER_W)], i_vm)

  iota = lax.iota(jnp.int32, LANES)
  ubufs = (ub0, ub1)
  ibufs = (ib0, ib1)
  usems = (su0, su1)
  isems = (si0, si1)

  def fire(j, k, slot):
    cj = pl.multiple_of((j >> 7) * TILE, TILE)
    pltpu.async_copy(
        user_hbm.at[:, pl.ds(cj, TILE)], ubufs[slot], usems[slot])
    ck = pl.multiple_of((k >> 7) * TILE, TILE)
    pltpu.async_copy(
        item_hbm.at[:, pl.ds(ck, TILE)], ibufs[slot], isems[slot])

  def consume(j, k, s, slot):
    pltpu.make_async_copy(
        user_hbm.at[:, pl.ds(0, TILE)], ubufs[slot], usems[slot]).wait()
    pltpu.make_async_copy(
        item_hbm.at[:, pl.ds(0, TILE)], ibufs[slot], isems[slot]).wait()
    lu_v = jnp.full((LANES,), j & (TILE - 1), jnp.int32)
    li_v = jnp.full((LANES,), k & (TILE - 1), jnp.int32)
    u0 = plsc.load_gather(ubufs[slot], [iota, lu_v])
    u1 = plsc.load_gather(ubufs[slot], [iota + HALF, lu_v])
    v0 = plsc.load_gather(ibufs[slot], [iota, li_v])
    v1 = plsc.load_gather(ibufs[slot], [iota + HALF, li_v])
    psum[s, pl.ds(0, LANES)] = u0 * v0 + u1 * v1

  @pl.loop(0, GROUPS)
  def _group(g):
    u16 = u_vm[pl.ds(g * LANES, LANES)]
    i16 = i_vm[pl.ds(g * LANES, LANES)]
    js = [u16[s] for s in range(LANES)]
    ks = [i16[s] for s in range(LANES)]
    fire(js[0], ks[0], 0)
    for s in range(LANES):
      if s + 1 < LANES:
        fire(js[s + 1], ks[s + 1], (s + 1) % 2)
      consume(js[s], ks[s], s, s % 2)
    acc = plsc.load_gather(psum, [iota, jnp.zeros((LANES,), jnp.int32)])
    for col in range(1, LANES):
      acc = acc + plsc.load_gather(
          psum, [iota, jnp.full((LANES,), col, jnp.int32)])
    out_v[pl.ds(g * LANES, LANES)] = acc

  pltpu.sync_copy(out_v, out_hbm.at[pl.ds(base, B_PER_W)])


@functools.partial(
    pl.kernel,
    out_type=jax.ShapeDtypeStruct((BATCH,), jnp.float32),
    mesh=plsc.VectorSubcoreMesh(
        core_axis_name="c", subcore_axis_name="s",
        num_cores=NUM_CORES, num_subcores=NUM_SUBCORES),
    compiler_params=pltpu.CompilerParams(
        needs_layout_passes=False, use_tc_tiling_on_sc=True),
    scratch_types=[
        pltpu.VMEM((B_PER_W,), jnp.int32),
        pltpu.VMEM((B_PER_W,), jnp.int32),
        pltpu.VMEM((EMBED_DIM, TILE), jnp.float32),
        pltpu.VMEM((EMBED_DIM, TILE), jnp.float32),
        pltpu.VMEM((EMBED_DIM, TILE), jnp.float32),
        pltpu.VMEM((EMBED_DIM, TILE), jnp.float32),
        pltpu.VMEM((LANES, SROW), jnp.float32),
        pltpu.VMEM((B_PER_W,), jnp.float32),
        pltpu.SemaphoreType.DMA,
        pltpu.SemaphoreType.DMA,
        pltpu.SemaphoreType.DMA,
        pltpu.SemaphoreType.DMA,
    ],
)
def _sc_dot(user_hbm, item_hbm, u_hbm, i_hbm, out_hbm,
            u_vm, i_vm, ub0, ub1, ib0, ib1, psum, out_v,
            su0, su1, si0, si1):
  _body(user_hbm, item_hbm, u_hbm, i_hbm, out_hbm,
        u_vm, i_vm, ub0, ub1, ib0, ib1, psum, out_v,
        su0, su1, si0, si1)


def kernel(u, i, user_table, item_table):
  return _sc_dot(user_table.T, item_table.T,
                 u.astype(jnp.int32), i.astype(jnp.int32))


# speedup vs baseline: 3.8669x; 1.3972x over previous
import functools

import jax
import jax.numpy as jnp
from jax import lax
from jax.experimental import pallas as pl
from jax.experimental.pallas import tpu as pltpu
from jax.experimental.pallas import tpu_sc as plsc

EMBED_DIM = 32
BATCH = 16384
TILE = 128

NUM_CORES = 2
NUM_SUBCORES = 16
NUM_WORKERS = NUM_CORES * NUM_SUBCORES
B_PER_W = BATCH // NUM_WORKERS
LANES = 16
HALF = LANES
GROUPS = B_PER_W // LANES
SROW = LANES + 1
DEPTH = 4


def _body(user_hbm, item_hbm, u_hbm, i_hbm, out_hbm,
          u_vm, i_vm, ub0, ub1, ub2, ub3, ib0, ib1, ib2, ib3, psum, out_v,
          su0, su1, su2, su3, si0, si1, si2, si3):
  wid = lax.axis_index("s") * NUM_CORES + lax.axis_index("c")
  base = wid * B_PER_W

  pltpu.sync_copy(u_hbm.at[pl.ds(base, B_PER_W)], u_vm)
  pltpu.sync_copy(i_hbm.at[pl.ds(base, B_PER_W)], i_vm)

  iota = lax.iota(jnp.int32, LANES)
  ubufs = (ub0, ub1, ub2, ub3)
  ibufs = (ib0, ib1, ib2, ib3)
  usems = (su0, su1, su2, su3)
  isems = (si0, si1, si2, si3)

  def fire(j, k, slot):
    cj = pl.multiple_of((j >> 7) * TILE, TILE)
    pltpu.async_copy(
        user_hbm.at[:, pl.ds(cj, TILE)], ubufs[slot], usems[slot])
    ck = pl.multiple_of((k >> 7) * TILE, TILE)
    pltpu.async_copy(
        item_hbm.at[:, pl.ds(ck, TILE)], ibufs[slot], isems[slot])

  def consume(j, k, s, slot):
    pltpu.make_async_copy(
        user_hbm.at[:, pl.ds(0, TILE)], ubufs[slot], usems[slot]).wait()
    pltpu.make_async_copy(
        item_hbm.at[:, pl.ds(0, TILE)], ibufs[slot], isems[slot]).wait()
    lu_v = jnp.full((LANES,), j & (TILE - 1), jnp.int32)
    li_v = jnp.full((LANES,), k & (TILE - 1), jnp.int32)
    u0 = plsc.load_gather(ubufs[slot], [iota, lu_v])
    u1 = plsc.load_gather(ubufs[slot], [iota + HALF, lu_v])
    v0 = plsc.load_gather(ibufs[slot], [iota, li_v])
    v1 = plsc.load_gather(ibufs[slot], [iota + HALF, li_v])
    psum[s, pl.ds(0, LANES)] = u0 * v0 + u1 * v1

  u16p = u_vm[pl.ds(0, LANES)]
  i16p = i_vm[pl.ds(0, LANES)]
  for s in range(DEPTH - 1):
    fire(u16p[s], i16p[s], s % DEPTH)

  @pl.loop(0, GROUPS)
  def _group(g):
    u16 = u_vm[pl.ds(g * LANES, LANES)]
    i16 = i_vm[pl.ds(g * LANES, LANES)]
    js = [u16[s] for s in range(LANES)]
    ks = [i16[s] for s in range(LANES)]
    for s in range(LANES):
      if s + DEPTH - 1 < LANES:
        fire(js[s + DEPTH - 1], ks[s + DEPTH - 1], (s + DEPTH - 1) % DEPTH)
      else:
        lane = s + DEPTH - 1 - LANES

        @pl.when(g < GROUPS - 1)
        def _():
          u16n = u_vm[pl.ds((g + 1) * LANES, LANES)]
          i16n = i_vm[pl.ds((g + 1) * LANES, LANES)]
          fire(u16n[lane], i16n[lane], (s + DEPTH - 1) % DEPTH)

      consume(js[s], ks[s], s, s % DEPTH)
    acc = plsc.load_gather(psum, [iota, jnp.zeros((LANES,), jnp.int32)])
    for col in range(1, LANES):
      acc = acc + plsc.load_gather(
          psum, [iota, jnp.full((LANES,), col, jnp.int32)])
    out_v[pl.ds(g * LANES, LANES)] = acc

  pltpu.sync_copy(out_v, out_hbm.at[pl.ds(base, B_PER_W)])


@functools.partial(
    pl.kernel,
    out_type=jax.ShapeDtypeStruct((BATCH,), jnp.float32),
    mesh=plsc.VectorSubcoreMesh(
        core_axis_name="c", subcore_axis_name="s",
        num_cores=NUM_CORES, num_subcores=NUM_SUBCORES),
    compiler_params=pltpu.CompilerParams(
        needs_layout_passes=False, use_tc_tiling_on_sc=True),
    scratch_types=[
        pltpu.VMEM((B_PER_W,), jnp.int32),
        pltpu.VMEM((B_PER_W,), jnp.int32),
    ] + [pltpu.VMEM((EMBED_DIM, TILE), jnp.float32)] * 8 + [
        pltpu.VMEM((LANES, SROW), jnp.float32),
        pltpu.VMEM((B_PER_W,), jnp.float32),
    ] + [pltpu.SemaphoreType.DMA] * 8,
)
def _sc_dot(user_hbm, item_hbm, u_hbm, i_hbm, out_hbm,
            u_vm, i_vm, ub0, ub1, ub2, ub3, ib0, ib1, ib2, ib3, psum, out_v,
            su0, su1, su2, su3, si0, si1, si2, si3):
  _body(user_hbm, item_hbm, u_hbm, i_hbm, out_hbm,
        u_vm, i_vm, ub0, ub1, ub2, ub3, ib0, ib1, ib2, ib3, psum, out_v,
        su0, su1, su2, su3, si0, si1, si2, si3)


def kernel(u, i, user_table, item_table):
  return _sc_dot(user_table.T, item_table.T,
                 u.astype(jnp.int32), i.astype(jnp.int32))
